# baseline (speedup 1.0000x reference)
import jax
import jax.numpy as jnp
from jax import lax
from jax.experimental import pallas as pl
from jax.experimental.pallas import tpu as pltpu
from jax.experimental.pallas import tpu_sc as plsc

N = 10000
E = 320000
F = 128
NC = 2
NS = 16
NW = NC * NS
NPAD = 10112
SROWS = NPAD // NS
DPAD = 10240

CH = 128
NCHUNK = E // CH
BASE = NCHUNK // NW
XTRA = NCHUNK - BASE * NW

_MESH = plsc.VectorSubcoreMesh(core_axis_name="c", subcore_axis_name="s")


def _init_accum(accum, z_hbm, row0):
    pltpu.sync_copy(z_hbm.at[pl.ds(row0, SROWS)],
                    accum.at[pl.ds(row0, SROWS)])


def _writeback(accum, po, row0, cid):
    pltpu.sync_copy(accum.at[pl.ds(row0, SROWS)],
                    po.at[pl.ds(cid * NPAD + row0, SROWS)])


def _worker_chunks(w):
    cnt = BASE + jnp.where(w < XTRA, 1, 0)
    s0 = w * BASE + jnp.minimum(w, XTRA)
    return cnt, s0


def _pipelined_chunks(cnt, s0, start, finish):
    start(s0, 0)

    def pair(j, _):
        i0 = s0 + 2 * j
        start(i0 + 1, 1)
        finish(i0, 0)

        @pl.when(2 * j + 2 < cnt)
        def _():
            start(i0 + 2, 0)

        finish(i0 + 1, 1)
        return 0

    lax.fori_loop(0, cnt // 2, pair, 0)

    @pl.when(cnt % 2 == 1)
    def _():
        finish(s0 + cnt - 1, 0)


PW = CH // 2
PSZ = (BASE + 1) * PW
EPACK = (BASE * NW + XTRA - 1) * PW + PSZ


def _pack_idx(idx):
    c2 = idx.reshape(E // CH, CH)
    p = (c2[:, :PW] | (c2[:, PW:] << 16)).reshape(-1)
    return jnp.pad(p, (0, EPACK - E // 2))


def _unpack_idx(packed, c, out_ref):
    for j in range(PW // 16):
        v = packed[pl.ds(c * PW + j * 16, 16)]
        out_ref[pl.ds(j * 16, 16)] = v & 0xFFFF
        out_ref[pl.ds(PW + j * 16, 16)] = jnp.right_shift(v, 16)


def _preload_packed(src_hbm, dst, s0):
    pltpu.sync_copy(src_hbm.at[pl.ds(s0 * PW, PSZ)], dst)


def _make_ea_kernel():
    scratch = [
        pltpu.VMEM((PSZ,), jnp.int32),
        pltpu.VMEM((PSZ,), jnp.int32),
        [pltpu.VMEM((CH,), jnp.int32) for _ in range(2)],
        [pltpu.VMEM((CH,), jnp.int32) for _ in range(2)],
        [pltpu.VMEM((CH, F), jnp.float32) for _ in range(2)],
        pltpu.VMEM_SHARED((NPAD, F), jnp.float32),
        [pltpu.SemaphoreType.DMA for _ in range(2)],
        [pltpu.SemaphoreType.DMA for _ in range(2)],
        [pltpu.SemaphoreType.DMA for _ in range(2)],
    ]

    def body(a_hbm, b_hbm, c_hbm, pc_hbm, pr_hbm, z_hbm, po,
             pcol, prow, idxc, idxr, bufab, accum, semb, sema, sems):
        cid = lax.axis_index("c")
        sid = lax.axis_index("s")
        w = sid * NC + cid
        row0 = sid * SROWS
        cnt, s0 = _worker_chunks(w)

        _init_accum(accum, z_hbm, row0)
        _preload_packed(pc_hbm, pcol, s0)
        _preload_packed(pr_hbm, prow, s0)
        plsc.subcore_barrier()

        def start(i, s):
            @pl.when(i >= s0 + 2)
            def _():
                pltpu.make_async_copy(bufab[s], accum.at[idxc[s]],
                                      sems[s]).wait()
            _unpack_idx(pcol, i - s0, idxc[s])
            _unpack_idx(prow, i - s0, idxr[s])
            pltpu.async_copy(c_hbm.at[pl.ds(i * CH, CH)], bufab[s], semb[s])

        def finish(i, s):
            pltpu.make_async_copy(c_hbm.at[pl.ds(i * CH, CH)], bufab[s],
                                  semb[s]).wait()
            pltpu.async_copy(a_hbm.at[idxc[s]], bufab[s], sema[s], add=True)
            pltpu.async_copy(b_hbm.at[idxr[s]], bufab[s], sema[s], add=True)
            pltpu.make_async_copy(a_hbm.at[idxc[s]], bufab[s],
                                  sema[s]).wait()
            pltpu.make_async_copy(b_hbm.at[idxr[s]], bufab[s],
                                  sema[s]).wait()

            @plsc.parallel_loop(0, CH, unroll=4)
            def _(r):
                for cc in range(F // 16):
                    sl = pl.ds(cc * 16, 16)
                    bufab[s][r, sl] = jnp.maximum(bufab[s][r, sl], 0.0)

            pltpu.async_copy(bufab[s], accum.at[idxc[s]], sems[s], add=True)

        _pipelined_chunks(cnt, s0, start, finish)
        for s in range(2):
            pltpu.make_async_copy(bufab[s], accum.at[idxc[s]],
                                  sems[s]).wait()

        plsc.subcore_barrier()
        _writeback(accum, po, row0, cid)

    return pl.kernel(
        body, out_type=jax.ShapeDtypeStruct((NC * NPAD, F), jnp.float32),
        mesh=_MESH, scratch_types=scratch)


def _make_hop_kernel():
    scratch = [
        pltpu.VMEM((PSZ,), jnp.int32),
        pltpu.VMEM((PSZ,), jnp.int32),
        [pltpu.VMEM((CH,), jnp.int32) for _ in range(2)],
        [pltpu.VMEM((CH,), jnp.int32) for _ in range(2)],
        [pltpu.VMEM((CH, F), jnp.float32) for _ in range(2)],
        pltpu.VMEM_SHARED((NPAD, F), jnp.float32),
        [pltpu.SemaphoreType.DMA for _ in range(2)],
        [pltpu.SemaphoreType.DMA for _ in range(2)],
    ]

    def body(t_hbm, pc_hbm, pr_hbm, z_hbm, po,
             pcol, prow, idxc, idxr, buf, accum, semb, sems):
        cid = lax.axis_index("c")
        sid = lax.axis_index("s")
        w = sid * NC + cid
        row0 = sid * SROWS
        cnt, s0 = _worker_chunks(w)

        _init_accum(accum, z_hbm, row0)
        _preload_packed(pc_hbm, pcol, s0)
        _preload_packed(pr_hbm, prow, s0)
        plsc.subcore_barrier()

        def start(i, s):
            @pl.when(i >= s0 + 2)
            def _():
                pltpu.make_async_copy(buf[s], accum.at[idxc[s]],
                                      sems[s]).wait()
            _unpack_idx(pcol, i - s0, idxc[s])
            _unpack_idx(prow, i - s0, idxr[s])
            pltpu.async_copy(t_hbm.at[idxr[s]], buf[s], semb[s])

        def finish(i, s):
            pltpu.make_async_copy(t_hbm.at[idxr[s]], buf[s], semb[s]).wait()
            pltpu.async_copy(buf[s], accum.at[idxc[s]], sems[s], add=True)

        _pipelined_chunks(cnt, s0, start, finish)
        for s in range(2):
            pltpu.make_async_copy(buf[s], accum.at[idxc[s]], sems[s]).wait()

        plsc.subcore_barrier()
        _writeback(accum, po, row0, cid)

    return pl.kernel(
        body, out_type=jax.ShapeDtypeStruct((NC * NPAD, F), jnp.float32),
        mesh=_MESH, scratch_types=scratch)


def _make_deg_kernel():
    DEGW = 128
    scratch = [
        pltpu.VMEM((PSZ,), jnp.int32),
        [pltpu.VMEM((CH,), jnp.int32) for _ in range(2)],
        pltpu.VMEM((CH, DEGW), jnp.float32),
        pltpu.VMEM_SHARED((NPAD, DEGW), jnp.float32),
        [pltpu.SemaphoreType.DMA for _ in range(2)],
    ]

    def body(pc_hbm, z_hbm, po, pcol, idxc, ones_v, dega, sems):
        cid = lax.axis_index("c")
        sid = lax.axis_index("s")
        w = sid * NC + cid
        row0 = sid * SROWS
        cnt, s0 = _worker_chunks(w)

        _init_accum(dega, z_hbm, row0)
        _preload_packed(pc_hbm, pcol, s0)

        def ob(r, _):
            for cc in range(DEGW // 16):
                ones_v[r, pl.ds(cc * 16, 16)] = jnp.ones((16,), jnp.float32)
            return 0
        lax.fori_loop(0, CH, ob, 0)

        plsc.subcore_barrier()

        def start(i, s):
            @pl.when(i >= s0 + 2)
            def _():
                pltpu.make_async_copy(ones_v, dega.at[idxc[s]],
                                      sems[s]).wait()
            _unpack_idx(pcol, i - s0, idxc[s])

        def finish(i, s):
            pltpu.async_copy(ones_v, dega.at[idxc[s]], sems[s], add=True)

        _pipelined_chunks(cnt, s0, start, finish)
        for s in range(2):
            pltpu.make_async_copy(ones_v, dega.at[idxc[s]], sems[s]).wait()

        plsc.subcore_barrier()
        _writeback(dega, po, row0, cid)

    return pl.kernel(
        body, out_type=jax.ShapeDtypeStruct((NC * NPAD, 128), jnp.float32),
        mesh=_MESH, scratch_types=scratch)


_ea_call = _make_ea_kernel()
_hop_call = _make_hop_kernel()
_deg_call = _make_deg_kernel()


BLK = 1264
EBLK = 2000


def _mm(x, w):
    p = w.shape[1]

    def body(x_ref, w_ref, o_ref):
        o_ref[...] = jnp.dot(x_ref[...], w_ref[...],
                             preferred_element_type=jnp.float32)

    return pl.pallas_call(
        body, grid=(NPAD // BLK,),
        in_specs=[pl.BlockSpec((BLK, F), lambda i: (i, 0)),
                  pl.BlockSpec((F, p), lambda i: (0, 0))],
        out_specs=pl.BlockSpec((BLK, p), lambda i: (i, 0)),
        out_shape=jax.ShapeDtypeStruct((NPAD, p), jnp.float32))(x, w)


def _cmat(ea, w, b):
    def body(e_ref, w_ref, b_ref, o_ref):
        o_ref[...] = jnp.dot(e_ref[...], w_ref[...],
                             preferred_element_type=jnp.float32) + b_ref[...]

    return pl.pallas_call(
        body, grid=(E // EBLK,),
        in_specs=[pl.BlockSpec((EBLK, 16), lambda i: (i, 0)),
                  pl.BlockSpec((16, F), lambda i: (0, 0)),
                  pl.BlockSpec((1, F), lambda i: (0, 0))],
        out_specs=pl.BlockSpec((EBLK, F), lambda i: (i, 0)),
        out_shape=jax.ShapeDtypeStruct((E, F), jnp.float32))(ea, w, b)


def _ea_post_tag_first(p, degp, w2, b2, w0):
    def body(p_ref, dp_ref, w2_ref, b2_ref, w0_ref, o0_ref, t0_ref, dg_ref,
             ds_ref):
        deg = dp_ref[0] + dp_ref[1]
        dis = jnp.where(deg > 0, lax.rsqrt(jnp.maximum(deg, 1e-12)), 0.0)
        s = p_ref[0] + p_ref[1]
        h = jnp.dot(s, w2_ref[...], preferred_element_type=jnp.float32)
        h = h + deg * b2_ref[...]
        o0_ref[...] = jnp.dot(h, w0_ref[...],
                              preferred_element_type=jnp.float32)
        t0_ref[...] = dis * h
        dg_ref[...] = deg
        ds_ref[...] = dis

    return pl.pallas_call(
        body, grid=(NPAD // BLK,),
        in_specs=[pl.BlockSpec((2, BLK, F), lambda i: (0, i, 0)),
                  pl.BlockSpec((2, BLK, 1), lambda i: (0, i, 0)),
                  pl.BlockSpec((F, F), lambda i: (0, 0)),
                  pl.BlockSpec((1, F), lambda i: (0, 0)),
                  pl.BlockSpec((F, F), lambda i: (0, 0))],
        out_specs=[pl.BlockSpec((BLK, F), lambda i: (i, 0)),
                   pl.BlockSpec((BLK, F), lambda i: (i, 0)),
                   pl.BlockSpec((BLK, 1), lambda i: (i, 0)),
                   pl.BlockSpec((BLK, 1), lambda i: (i, 0))],
        out_shape=[jax.ShapeDtypeStruct((NPAD, F), jnp.float32),
                   jax.ShapeDtypeStruct((NPAD, F), jnp.float32),
                   jax.ShapeDtypeStruct((NPAD, 1), jnp.float32),
                   jax.ShapeDtypeStruct((NPAD, 1), jnp.float32)])(
                       p, degp, w2, b2, w0)


def _ea_post_tag_next(p, deg, dis, w2, b2, w0):
    def body(p_ref, dg_ref, ds_ref, w2_ref, b2_ref, w0_ref, o0_ref, t0_ref):
        s = p_ref[0] + p_ref[1]
        h = jnp.dot(s, w2_ref[...], preferred_element_type=jnp.float32)
        h = h + dg_ref[...] * b2_ref[...]
        o0_ref[...] = jnp.dot(h, w0_ref[...],
                              preferred_element_type=jnp.float32)
        t0_ref[...] = ds_ref[...] * h

    return pl.pallas_call(
        body, grid=(NPAD // BLK,),
        in_specs=[pl.BlockSpec((2, BLK, F), lambda i: (0, i, 0)),
                  pl.BlockSpec((BLK, 1), lambda i: (i, 0)),
                  pl.BlockSpec((BLK, 1), lambda i: (i, 0)),
                  pl.BlockSpec((F, F), lambda i: (0, 0)),
                  pl.BlockSpec((1, F), lambda i: (0, 0)),
                  pl.BlockSpec((F, F), lambda i: (0, 0))],
        out_specs=[pl.BlockSpec((BLK, F), lambda i: (i, 0)),
                   pl.BlockSpec((BLK, F), lambda i: (i, 0))],
        out_shape=[jax.ShapeDtypeStruct((NPAD, F), jnp.float32),
                   jax.ShapeDtypeStruct((NPAD, F), jnp.float32)])(
                       p, deg, dis, w2, b2, w0)


def _hopmix(p, dis, wk, outprev):
    def body(p_ref, ds_ref, wk_ref, op_ref, o_ref, t_ref):
        u = ds_ref[...] * (p_ref[0] + p_ref[1])
        o_ref[...] = op_ref[...] + jnp.dot(
            u, wk_ref[...], preferred_element_type=jnp.float32)
        t_ref[...] = ds_ref[...] * u

    return pl.pallas_call(
        body, grid=(NPAD // BLK,),
        in_specs=[pl.BlockSpec((2, BLK, F), lambda i: (0, i, 0)),
                  pl.BlockSpec((BLK, 1), lambda i: (i, 0)),
                  pl.BlockSpec((F, F), lambda i: (0, 0)),
                  pl.BlockSpec((BLK, F), lambda i: (i, 0))],
        out_specs=[pl.BlockSpec((BLK, F), lambda i: (i, 0)),
                   pl.BlockSpec((BLK, F), lambda i: (i, 0))],
        out_shape=[jax.ShapeDtypeStruct((NPAD, F), jnp.float32),
                   jax.ShapeDtypeStruct((NPAD, F), jnp.float32)])(
                       p, dis, wk, outprev)


def _hopmix_last_relu_mm(p, dis, wk, outprev, bc, wab):
    def body(p_ref, ds_ref, wk_ref, op_ref, bc_ref, wab_ref, ab_ref):
        u = ds_ref[...] * (p_ref[0] + p_ref[1])
        o = op_ref[...] + jnp.dot(u, wk_ref[...],
                                  preferred_element_type=jnp.float32)
        r = jnp.maximum(o + bc_ref[...], 0.0)
        ab_ref[...] = jnp.dot(r, wab_ref[...],
                              preferred_element_type=jnp.float32)

    return pl.pallas_call(
        body, grid=(NPAD // BLK,),
        in_specs=[pl.BlockSpec((2, BLK, F), lambda i: (0, i, 0)),
                  pl.BlockSpec((BLK, 1), lambda i: (i, 0)),
                  pl.BlockSpec((F, F), lambda i: (0, 0)),
                  pl.BlockSpec((BLK, F), lambda i: (i, 0)),
                  pl.BlockSpec((1, F), lambda i: (0, 0)),
                  pl.BlockSpec((F, 2 * F), lambda i: (0, 0))],
        out_specs=pl.BlockSpec((BLK, 2 * F), lambda i: (i, 0)),
        out_shape=jax.ShapeDtypeStruct((NPAD, 2 * F), jnp.float32))(
            p, dis, wk, outprev, bc, wab)


def _hopmix_bias(p, dis, wk, outprev, bc):
    def body(p_ref, ds_ref, wk_ref, op_ref, bc_ref, o_ref):
        u = ds_ref[...] * (p_ref[0] + p_ref[1])
        o_ref[...] = op_ref[...] + jnp.dot(
            u, wk_ref[...], preferred_element_type=jnp.float32) + bc_ref[...]

    return pl.pallas_call(
        body, grid=(NPAD // BLK,),
        in_specs=[pl.BlockSpec((2, BLK, F), lambda i: (0, i, 0)),
                  pl.BlockSpec((BLK, 1), lambda i: (i, 0)),
                  pl.BlockSpec((F, F), lambda i: (0, 0)),
                  pl.BlockSpec((BLK, F), lambda i: (i, 0)),
                  pl.BlockSpec((1, F), lambda i: (0, 0))],
        out_specs=pl.BlockSpec((BLK, F), lambda i: (i, 0)),
        out_shape=jax.ShapeDtypeStruct((NPAD, F), jnp.float32))(
            p, dis, wk, outprev, bc)


def kernel(x, edge_index, edge_attr, ea1_W1, ea1_b1, ea1_W2, ea1_b2,
           ea2_W1, ea2_b1, ea2_W2, ea2_b2, conv0_W, conv0_b,
           conv1_W, conv1_b, conv2_W, conv2_b):
    pr = _pack_idx(edge_index[0])
    pc = _pack_idx(edge_index[1])
    xf = jnp.pad(x[:, 4:4 + F], ((0, NPAD - N), (0, 0)))

    w1ab_1 = jnp.concatenate([ea1_W1[:F], ea1_W1[F:2 * F]], axis=1)
    w1ab_2 = jnp.concatenate([ea2_W1[:F], ea2_W1[F:2 * F]], axis=1)
    c1 = _cmat(edge_attr, ea1_W1[2 * F:], ea1_b1.reshape(1, F))
    c2 = _cmat(edge_attr, ea2_W1[2 * F:], ea2_b1.reshape(1, F))
    zz = jnp.zeros((NPAD, F), jnp.float32)
    degp = _deg_call(pc, zz)

    ab = _mm(xf, w1ab_1)
    po = _ea_call(ab[:, :F], ab[:, F:], c1, pc, pr, zz)
    outp, t, deg, dis = _ea_post_tag_first(
        po.reshape(NC, NPAD, F), degp.reshape(NC, NPAD, F)[:, :, 0:1],
        ea1_W2, ea1_b2.reshape(1, F), conv0_W[0])
    for k in (1, 2):
        pk = _hop_call(t, pc, pr, zz).reshape(NC, NPAD, F)
        outp, t = _hopmix(pk, dis, conv0_W[k], outp)
    pk = _hop_call(t, pc, pr, zz).reshape(NC, NPAD, F)
    ab = _hopmix_last_relu_mm(pk, dis, conv0_W[3], outp,
                              conv0_b.reshape(1, F), w1ab_2)

    po = _ea_call(ab[:, :F], ab[:, F:], c2, pc, pr, zz)
    outp, t = _ea_post_tag_next(po.reshape(NC, NPAD, F), deg, dis,
                                ea2_W2, ea2_b2.reshape(1, F), conv1_W[0])
    for k in (1, 2):
        pk = _hop_call(t, pc, pr, zz).reshape(NC, NPAD, F)
        outp, t = _hopmix(pk, dis, conv1_W[k], outp)
    pk = _hop_call(t, pc, pr, zz).reshape(NC, NPAD, F)
    ab = _hopmix_last_relu_mm(pk, dis, conv1_W[3], outp,
                              conv1_b.reshape(1, F), w1ab_2)

    po = _ea_call(ab[:, :F], ab[:, F:], c2, pc, pr, zz)
    outp, t = _ea_post_tag_next(po.reshape(NC, NPAD, F), deg, dis,
                                ea2_W2, ea2_b2.reshape(1, F), conv2_W[0])
    for k in (1, 2):
        pk = _hop_call(t, pc, pr, zz).reshape(NC, NPAD, F)
        outp, t = _hopmix(pk, dis, conv2_W[k], outp)
    pk = _hop_call(t, pc, pr, zz).reshape(NC, NPAD, F)
    out = _hopmix_bias(pk, dis, conv2_W[3], outp, conv2_b.reshape(1, F))
    return out[:N]


# speedup vs baseline: 1.0113x; 1.0113x over previous
import jax
import jax.numpy as jnp
from jax import lax
from jax.experimental import pallas as pl
from jax.experimental.pallas import tpu as pltpu
from jax.experimental.pallas import tpu_sc as plsc

N = 10000
E = 320000
F = 128
NC = 2
NS = 16
NW = NC * NS
NPAD = 10112
SROWS = NPAD // NS
DPAD = 10240

CH = 128
NCHUNK = E // CH
BASE = NCHUNK // NW
XTRA = NCHUNK - BASE * NW

_WB = [(0, 128), (128, 128), (256, 128), (384, 128), (512, 120)]

_MESH = plsc.VectorSubcoreMesh(core_axis_name="c", subcore_axis_name="s")


def _zero_ref(ref, rows, width):
    def body(r, _):
        for cc in range(width // 16):
            ref[r, pl.ds(cc * 16, 16)] = jnp.zeros((16,), jnp.float32)
        return 0
    lax.fori_loop(0, rows, body, 0)


def _init_accum(accum, stage, row0, width):
    _zero_ref(stage, 128, width)
    for off, sz in _WB:
        pltpu.sync_copy(stage.at[pl.ds(0, sz)],
                        accum.at[pl.ds(row0 + off, sz)])


def _writeback(accum, stage, po, row0, cid):
    for off, sz in _WB:
        pltpu.sync_copy(accum.at[pl.ds(row0 + off, sz)],
                        stage.at[pl.ds(0, sz)])
        pltpu.sync_copy(stage.at[pl.ds(0, sz)],
                        po.at[pl.ds(cid * NPAD + row0 + off, sz)])


def _worker_chunks(w):
    cnt = BASE + jnp.where(w < XTRA, 1, 0)
    s0 = w * BASE + jnp.minimum(w, XTRA)
    return cnt, s0


def _pipelined_chunks(cnt, s0, start, finish):
    start(s0, 0)

    def pair(j, _):
        i0 = s0 + 2 * j
        start(i0 + 1, 1)
        finish(i0, 0)

        @pl.when(2 * j + 2 < cnt)
        def _():
            start(i0 + 2, 0)

        finish(i0 + 1, 1)
        return 0

    lax.fori_loop(0, cnt // 2, pair, 0)

    @pl.when(cnt % 2 == 1)
    def _():
        finish(s0 + cnt - 1, 0)


PW = CH // 2
PSZ = (BASE + 1) * PW
EPACK = (BASE * NW + XTRA - 1) * PW + PSZ


def _pack_idx(idx):
    c2 = idx.reshape(E // CH, CH)
    p = (c2[:, :PW] | (c2[:, PW:] << 16)).reshape(-1)
    return jnp.pad(p, (0, EPACK - E // 2))


def _unpack_idx(packed, c, out_ref):
    for j in range(PW // 16):
        v = packed[pl.ds(c * PW + j * 16, 16)]
        out_ref[pl.ds(j * 16, 16)] = v & 0xFFFF
        out_ref[pl.ds(PW + j * 16, 16)] = jnp.right_shift(v, 16)


def _preload_packed(src_hbm, dst, s0):
    pltpu.sync_copy(src_hbm.at[pl.ds(s0 * PW, PSZ)], dst)


def _make_ea_kernel():
    scratch = [
        pltpu.VMEM((PSZ,), jnp.int32),
        pltpu.VMEM((PSZ,), jnp.int32),
        [pltpu.VMEM((CH,), jnp.int32) for _ in range(2)],
        [pltpu.VMEM((CH,), jnp.int32) for _ in range(2)],
        [pltpu.VMEM((CH, F), jnp.float32) for _ in range(2)],
        pltpu.VMEM_SHARED((NPAD, F), jnp.float32),
        [pltpu.SemaphoreType.DMA for _ in range(2)],
        [pltpu.SemaphoreType.DMA for _ in range(2)],
        [pltpu.SemaphoreType.DMA for _ in range(2)],
    ]

    def body(a_hbm, b_hbm, c_hbm, pc_hbm, pr_hbm, po,
             pcol, prow, idxc, idxr, bufab, accum, semb, sema, sems):
        cid = lax.axis_index("c")
        sid = lax.axis_index("s")
        w = sid * NC + cid
        row0 = sid * SROWS
        cnt, s0 = _worker_chunks(w)

        _init_accum(accum, bufab[0], row0, F)
        _preload_packed(pc_hbm, pcol, s0)
        _preload_packed(pr_hbm, prow, s0)
        plsc.subcore_barrier()

        def start(i, s):
            @pl.when(i >= s0 + 2)
            def _():
                pltpu.make_async_copy(bufab[s], accum.at[idxc[s]],
                                      sems[s]).wait()
            _unpack_idx(pcol, i - s0, idxc[s])
            _unpack_idx(prow, i - s0, idxr[s])
            pltpu.async_copy(c_hbm.at[pl.ds(i * CH, CH)], bufab[s], semb[s])

        def finish(i, s):
            pltpu.make_async_copy(c_hbm.at[pl.ds(i * CH, CH)], bufab[s],
                                  semb[s]).wait()
            pltpu.async_copy(a_hbm.at[idxc[s]], bufab[s], sema[s], add=True)
            pltpu.async_copy(b_hbm.at[idxr[s]], bufab[s], sema[s], add=True)
            pltpu.make_async_copy(a_hbm.at[idxc[s]], bufab[s],
                                  sema[s]).wait()
            pltpu.make_async_copy(b_hbm.at[idxr[s]], bufab[s],
                                  sema[s]).wait()

            @plsc.parallel_loop(0, CH, unroll=4)
            def _(r):
                for cc in range(F // 16):
                    sl = pl.ds(cc * 16, 16)
                    bufab[s][r, sl] = jnp.maximum(bufab[s][r, sl], 0.0)

            pltpu.async_copy(bufab[s], accum.at[idxc[s]], sems[s], add=True)

        _pipelined_chunks(cnt, s0, start, finish)
        for s in range(2):
            pltpu.make_async_copy(bufab[s], accum.at[idxc[s]],
                                  sems[s]).wait()

        plsc.subcore_barrier()
        _writeback(accum, bufab[0], po, row0, cid)

    return pl.kernel(
        body, out_type=jax.ShapeDtypeStruct((NC * NPAD, F), jnp.float32),
        mesh=_MESH, scratch_types=scratch)


def _make_hop_kernel():
    scratch = [
        pltpu.VMEM((PSZ,), jnp.int32),
        pltpu.VMEM((PSZ,), jnp.int32),
        [pltpu.VMEM((CH,), jnp.int32) for _ in range(2)],
        [pltpu.VMEM((CH,), jnp.int32) for _ in range(2)],
        [pltpu.VMEM((CH, F), jnp.float32) for _ in range(2)],
        pltpu.VMEM_SHARED((NPAD, F), jnp.float32),
        [pltpu.SemaphoreType.DMA for _ in range(2)],
        [pltpu.SemaphoreType.DMA for _ in range(2)],
    ]

    def body(t_hbm, pc_hbm, pr_hbm, po,
             pcol, prow, idxc, idxr, buf, accum, semb, sems):
        cid = lax.axis_index("c")
        sid = lax.axis_index("s")
        w = sid * NC + cid
        row0 = sid * SROWS
        cnt, s0 = _worker_chunks(w)

        _init_accum(accum, buf[0], row0, F)
        _preload_packed(pc_hbm, pcol, s0)
        _preload_packed(pr_hbm, prow, s0)
        plsc.subcore_barrier()

        def start(i, s):
            @pl.when(i >= s0 + 2)
            def _():
                pltpu.make_async_copy(buf[s], accum.at[idxc[s]],
                                      sems[s]).wait()
            _unpack_idx(pcol, i - s0, idxc[s])
            _unpack_idx(prow, i - s0, idxr[s])
            pltpu.async_copy(t_hbm.at[idxr[s]], buf[s], semb[s])

        def finish(i, s):
            pltpu.make_async_copy(t_hbm.at[idxr[s]], buf[s], semb[s]).wait()
            pltpu.async_copy(buf[s], accum.at[idxc[s]], sems[s], add=True)

        _pipelined_chunks(cnt, s0, start, finish)
        for s in range(2):
            pltpu.make_async_copy(buf[s], accum.at[idxc[s]], sems[s]).wait()

        plsc.subcore_barrier()
        _writeback(accum, buf[0], po, row0, cid)

    return pl.kernel(
        body, out_type=jax.ShapeDtypeStruct((NC * NPAD, F), jnp.float32),
        mesh=_MESH, scratch_types=scratch)


def _make_deg_kernel():
    DEGW = 128
    scratch = [
        pltpu.VMEM((PSZ,), jnp.int32),
        [pltpu.VMEM((CH,), jnp.int32) for _ in range(2)],
        pltpu.VMEM((CH, DEGW), jnp.float32),
        pltpu.VMEM_SHARED((NPAD, DEGW), jnp.float32),
        [pltpu.SemaphoreType.DMA for _ in range(2)],
    ]

    def body(pc_hbm, po, pcol, idxc, ones_v, dega, sems):
        cid = lax.axis_index("c")
        sid = lax.axis_index("s")
        w = sid * NC + cid
        row0 = sid * SROWS
        cnt, s0 = _worker_chunks(w)

        _init_accum(dega, ones_v, row0, DEGW)
        _preload_packed(pc_hbm, pcol, s0)

        def ob(r, _):
            for cc in range(DEGW // 16):
                ones_v[r, pl.ds(cc * 16, 16)] = jnp.ones((16,), jnp.float32)
            return 0
        lax.fori_loop(0, CH, ob, 0)

        plsc.subcore_barrier()

        def start(i, s):
            @pl.when(i >= s0 + 2)
            def _():
                pltpu.make_async_copy(ones_v, dega.at[idxc[s]],
                                      sems[s]).wait()
            _unpack_idx(pcol, i - s0, idxc[s])

        def finish(i, s):
            pltpu.async_copy(ones_v, dega.at[idxc[s]], sems[s], add=True)

        _pipelined_chunks(cnt, s0, start, finish)
        for s in range(2):
            pltpu.make_async_copy(ones_v, dega.at[idxc[s]], sems[s]).wait()

        plsc.subcore_barrier()
        _writeback(dega, ones_v, po, row0, cid)

    return pl.kernel(
        body, out_type=jax.ShapeDtypeStruct((NC * NPAD, 128), jnp.float32),
        mesh=_MESH, scratch_types=scratch)


_ea_call = _make_ea_kernel()
_hop_call = _make_hop_kernel()
_deg_call = _make_deg_kernel()


BLK = 1264
EBLK = 2000


def _mm(x, w):
    p = w.shape[1]

    def body(x_ref, w_ref, o_ref):
        o_ref[...] = jnp.dot(x_ref[...], w_ref[...],
                             preferred_element_type=jnp.float32)

    return pl.pallas_call(
        body, grid=(NPAD // BLK,),
        in_specs=[pl.BlockSpec((BLK, F), lambda i: (i, 0)),
                  pl.BlockSpec((F, p), lambda i: (0, 0))],
        out_specs=pl.BlockSpec((BLK, p), lambda i: (i, 0)),
        out_shape=jax.ShapeDtypeStruct((NPAD, p), jnp.float32))(x, w)


def _cmat(ea, w, b):
    def body(e_ref, w_ref, b_ref, o_ref):
        o_ref[...] = jnp.dot(e_ref[...], w_ref[...],
                             preferred_element_type=jnp.float32) + b_ref[...]

    return pl.pallas_call(
        body, grid=(E // EBLK,),
        in_specs=[pl.BlockSpec((EBLK, 16), lambda i: (i, 0)),
                  pl.BlockSpec((16, F), lambda i: (0, 0)),
                  pl.BlockSpec((1, F), lambda i: (0, 0))],
        out_specs=pl.BlockSpec((EBLK, F), lambda i: (i, 0)),
        out_shape=jax.ShapeDtypeStruct((E, F), jnp.float32))(ea, w, b)


def _ea_post_tag_first(p, degp, w2, b2, w0):
    def body(p_ref, dp_ref, w2_ref, b2_ref, w0_ref, o0_ref, t0_ref, dg_ref,
             ds_ref):
        deg = dp_ref[0] + dp_ref[1]
        dis = jnp.where(deg > 0, lax.rsqrt(jnp.maximum(deg, 1e-12)), 0.0)
        s = p_ref[0] + p_ref[1]
        h = jnp.dot(s, w2_ref[...], preferred_element_type=jnp.float32)
        h = h + deg * b2_ref[...]
        o0_ref[...] = jnp.dot(h, w0_ref[...],
                              preferred_element_type=jnp.float32)
        t0_ref[...] = dis * h
        dg_ref[...] = deg
        ds_ref[...] = dis

    return pl.pallas_call(
        body, grid=(NPAD // BLK,),
        in_specs=[pl.BlockSpec((2, BLK, F), lambda i: (0, i, 0)),
                  pl.BlockSpec((2, BLK, 1), lambda i: (0, i, 0)),
                  pl.BlockSpec((F, F), lambda i: (0, 0)),
                  pl.BlockSpec((1, F), lambda i: (0, 0)),
                  pl.BlockSpec((F, F), lambda i: (0, 0))],
        out_specs=[pl.BlockSpec((BLK, F), lambda i: (i, 0)),
                   pl.BlockSpec((BLK, F), lambda i: (i, 0)),
                   pl.BlockSpec((BLK, 1), lambda i: (i, 0)),
                   pl.BlockSpec((BLK, 1), lambda i: (i, 0))],
        out_shape=[jax.ShapeDtypeStruct((NPAD, F), jnp.float32),
                   jax.ShapeDtypeStruct((NPAD, F), jnp.float32),
                   jax.ShapeDtypeStruct((NPAD, 1), jnp.float32),
                   jax.ShapeDtypeStruct((NPAD, 1), jnp.float32)])(
                       p, degp, w2, b2, w0)


def _ea_post_tag_next(p, deg, dis, w2, b2, w0):
    def body(p_ref, dg_ref, ds_ref, w2_ref, b2_ref, w0_ref, o0_ref, t0_ref):
        s = p_ref[0] + p_ref[1]
        h = jnp.dot(s, w2_ref[...], preferred_element_type=jnp.float32)
        h = h + dg_ref[...] * b2_ref[...]
        o0_ref[...] = jnp.dot(h, w0_ref[...],
                              preferred_element_type=jnp.float32)
        t0_ref[...] = ds_ref[...] * h

    return pl.pallas_call(
        body, grid=(NPAD // BLK,),
        in_specs=[pl.BlockSpec((2, BLK, F), lambda i: (0, i, 0)),
                  pl.BlockSpec((BLK, 1), lambda i: (i, 0)),
                  pl.BlockSpec((BLK, 1), lambda i: (i, 0)),
                  pl.BlockSpec((F, F), lambda i: (0, 0)),
                  pl.BlockSpec((1, F), lambda i: (0, 0)),
                  pl.BlockSpec((F, F), lambda i: (0, 0))],
        out_specs=[pl.BlockSpec((BLK, F), lambda i: (i, 0)),
                   pl.BlockSpec((BLK, F), lambda i: (i, 0))],
        out_shape=[jax.ShapeDtypeStruct((NPAD, F), jnp.float32),
                   jax.ShapeDtypeStruct((NPAD, F), jnp.float32)])(
                       p, deg, dis, w2, b2, w0)


def _hopmix(p, dis, wk, outprev):
    def body(p_ref, ds_ref, wk_ref, op_ref, o_ref, t_ref):
        u = ds_ref[...] * (p_ref[0] + p_ref[1])
        o_ref[...] = op_ref[...] + jnp.dot(
            u, wk_ref[...], preferred_element_type=jnp.float32)
        t_ref[...] = ds_ref[...] * u

    return pl.pallas_call(
        body, grid=(NPAD // BLK,),
        in_specs=[pl.BlockSpec((2, BLK, F), lambda i: (0, i, 0)),
                  pl.BlockSpec((BLK, 1), lambda i: (i, 0)),
                  pl.BlockSpec((F, F), lambda i: (0, 0)),
                  pl.BlockSpec((BLK, F), lambda i: (i, 0))],
        out_specs=[pl.BlockSpec((BLK, F), lambda i: (i, 0)),
                   pl.BlockSpec((BLK, F), lambda i: (i, 0))],
        out_shape=[jax.ShapeDtypeStruct((NPAD, F), jnp.float32),
                   jax.ShapeDtypeStruct((NPAD, F), jnp.float32)])(
                       p, dis, wk, outprev)


def _hopmix_last_relu_mm(p, dis, wk, outprev, bc, wab):
    def body(p_ref, ds_ref, wk_ref, op_ref, bc_ref, wab_ref, ab_ref):
        u = ds_ref[...] * (p_ref[0] + p_ref[1])
        o = op_ref[...] + jnp.dot(u, wk_ref[...],
                                  preferred_element_type=jnp.float32)
        r = jnp.maximum(o + bc_ref[...], 0.0)
        ab_ref[...] = jnp.dot(r, wab_ref[...],
                              preferred_element_type=jnp.float32)

    return pl.pallas_call(
        body, grid=(NPAD // BLK,),
        in_specs=[pl.BlockSpec((2, BLK, F), lambda i: (0, i, 0)),
                  pl.BlockSpec((BLK, 1), lambda i: (i, 0)),
                  pl.BlockSpec((F, F), lambda i: (0, 0)),
                  pl.BlockSpec((BLK, F), lambda i: (i, 0)),
                  pl.BlockSpec((1, F), lambda i: (0, 0)),
                  pl.BlockSpec((F, 2 * F), lambda i: (0, 0))],
        out_specs=pl.BlockSpec((BLK, 2 * F), lambda i: (i, 0)),
        out_shape=jax.ShapeDtypeStruct((NPAD, 2 * F), jnp.float32))(
            p, dis, wk, outprev, bc, wab)


def _hopmix_bias(p, dis, wk, outprev, bc):
    def body(p_ref, ds_ref, wk_ref, op_ref, bc_ref, o_ref):
        u = ds_ref[...] * (p_ref[0] + p_ref[1])
        o_ref[...] = op_ref[...] + jnp.dot(
            u, wk_ref[...], preferred_element_type=jnp.float32) + bc_ref[...]

    return pl.pallas_call(
        body, grid=(NPAD // BLK,),
        in_specs=[pl.BlockSpec((2, BLK, F), lambda i: (0, i, 0)),
                  pl.BlockSpec((BLK, 1), lambda i: (i, 0)),
                  pl.BlockSpec((F, F), lambda i: (0, 0)),
                  pl.BlockSpec((BLK, F), lambda i: (i, 0)),
                  pl.BlockSpec((1, F), lambda i: (0, 0))],
        out_specs=pl.BlockSpec((BLK, F), lambda i: (i, 0)),
        out_shape=jax.ShapeDtypeStruct((NPAD, F), jnp.float32))(
            p, dis, wk, outprev, bc)


def kernel(x, edge_index, edge_attr, ea1_W1, ea1_b1, ea1_W2, ea1_b2,
           ea2_W1, ea2_b1, ea2_W2, ea2_b2, conv0_W, conv0_b,
           conv1_W, conv1_b, conv2_W, conv2_b):
    pr = _pack_idx(edge_index[0])
    pc = _pack_idx(edge_index[1])
    xf = jnp.pad(x[:, 4:4 + F], ((0, NPAD - N), (0, 0)))

    w1ab_1 = jnp.concatenate([ea1_W1[:F], ea1_W1[F:2 * F]], axis=1)
    w1ab_2 = jnp.concatenate([ea2_W1[:F], ea2_W1[F:2 * F]], axis=1)
    c1 = _cmat(edge_attr, ea1_W1[2 * F:], ea1_b1.reshape(1, F))
    c2 = _cmat(edge_attr, ea2_W1[2 * F:], ea2_b1.reshape(1, F))
    degp = _deg_call(pc)

    ab = _mm(xf, w1ab_1)
    po = _ea_call(ab[:, :F], ab[:, F:], c1, pc, pr)
    outp, t, deg, dis = _ea_post_tag_first(
        po.reshape(NC, NPAD, F), degp.reshape(NC, NPAD, F)[:, :, 0:1],
        ea1_W2, ea1_b2.reshape(1, F), conv0_W[0])
    for k in (1, 2):
        pk = _hop_call(t, pc, pr).reshape(NC, NPAD, F)
        outp, t = _hopmix(pk, dis, conv0_W[k], outp)
    pk = _hop_call(t, pc, pr).reshape(NC, NPAD, F)
    ab = _hopmix_last_relu_mm(pk, dis, conv0_W[3], outp,
                              conv0_b.reshape(1, F), w1ab_2)

    po = _ea_call(ab[:, :F], ab[:, F:], c2, pc, pr)
    outp, t = _ea_post_tag_next(po.reshape(NC, NPAD, F), deg, dis,
                                ea2_W2, ea2_b2.reshape(1, F), conv1_W[0])
    for k in (1, 2):
        pk = _hop_call(t, pc, pr).reshape(NC, NPAD, F)
        outp, t = _hopmix(pk, dis, conv1_W[k], outp)
    pk = _hop_call(t, pc, pr).reshape(NC, NPAD, F)
    ab = _hopmix_last_relu_mm(pk, dis, conv1_W[3], outp,
                              conv1_b.reshape(1, F), w1ab_2)

    po = _ea_call(ab[:, :F], ab[:, F:], c2, pc, pr)
    outp, t = _ea_post_tag_next(po.reshape(NC, NPAD, F), deg, dis,
                                ea2_W2, ea2_b2.reshape(1, F), conv2_W[0])
    for k in (1, 2):
        pk = _hop_call(t, pc, pr).reshape(NC, NPAD, F)
        outp, t = _hopmix(pk, dis, conv2_W[k], outp)
    pk = _hop_call(t, pc, pr).reshape(NC, NPAD, F)
    out = _hopmix_bias(pk, dis, conv2_W[3], outp, conv2_b.reshape(1, F))
    return out[:N]


# speedup vs baseline: 1.1171x; 1.1045x over previous
import jax
import jax.numpy as jnp
from jax import lax
from jax.experimental import pallas as pl
from jax.experimental.pallas import tpu as pltpu
from jax.experimental.pallas import tpu_sc as plsc

N = 10000
E = 320000
F = 128
NC = 2
NS = 16
NW = NC * NS
NPAD = 10112
SROWS = NPAD // NS
DPAD = 10240

CH = 128
NCHUNK = E // CH
BASE = NCHUNK // NW
XTRA = NCHUNK - BASE * NW

_WB = [(0, 128), (128, 128), (256, 128), (384, 128), (512, 120)]

_MESH = plsc.VectorSubcoreMesh(core_axis_name="c", subcore_axis_name="s")


def _zero_ref(ref, rows, width):
    def body(r, _):
        for cc in range(width // 16):
            ref[r, pl.ds(cc * 16, 16)] = jnp.zeros((16,), jnp.float32)
        return 0
    lax.fori_loop(0, rows, body, 0)


def _init_accum(accum, stage, row0, width):
    _zero_ref(stage, 128, width)
    for off, sz in _WB:
        pltpu.sync_copy(stage.at[pl.ds(0, sz)],
                        accum.at[pl.ds(row0 + off, sz)])


def _writeback(accum, stage, po, row0, cid):
    for off, sz in _WB:
        pltpu.sync_copy(accum.at[pl.ds(row0 + off, sz)],
                        stage.at[pl.ds(0, sz)])
        pltpu.sync_copy(stage.at[pl.ds(0, sz)],
                        po.at[pl.ds(cid * NPAD + row0 + off, sz)])


def _worker_chunks(w):
    cnt = BASE + jnp.where(w < XTRA, 1, 0)
    s0 = w * BASE + jnp.minimum(w, XTRA)
    return cnt, s0


def _pipelined_chunks(cnt, s0, start, finish):
    start(s0, 0)

    def pair(j, _):
        i0 = s0 + 2 * j
        start(i0 + 1, 1)
        finish(i0, 0)

        @pl.when(2 * j + 2 < cnt)
        def _():
            start(i0 + 2, 0)

        finish(i0 + 1, 1)
        return 0

    lax.fori_loop(0, cnt // 2, pair, 0)

    @pl.when(cnt % 2 == 1)
    def _():
        finish(s0 + cnt - 1, 0)


def _pipelined_chunks3(cnt, s0, start, issue, complete):
    start(s0, 0)

    def pair(j, _):
        i0 = s0 + 2 * j
        start(i0 + 1, 1)
        issue(i0, 0)
        issue(i0 + 1, 1)
        complete(i0, 0)

        @pl.when(2 * j + 2 < cnt)
        def _():
            start(i0 + 2, 0)

        complete(i0 + 1, 1)
        return 0

    lax.fori_loop(0, cnt // 2, pair, 0)

    @pl.when(cnt % 2 == 1)
    def _():
        issue(s0 + cnt - 1, 0)
        complete(s0 + cnt - 1, 0)


PW = CH // 2
PSZ = (BASE + 1) * PW
EPACK = (BASE * NW + XTRA - 1) * PW + PSZ


def _pack_idx(idx):
    c2 = idx.reshape(E // CH, CH)
    p = (c2[:, :PW] | (c2[:, PW:] << 16)).reshape(-1)
    return jnp.pad(p, (0, EPACK - E // 2))


def _unpack_idx(packed, c, out_ref):
    for j in range(PW // 16):
        v = packed[pl.ds(c * PW + j * 16, 16)]
        out_ref[pl.ds(j * 16, 16)] = v & 0xFFFF
        out_ref[pl.ds(PW + j * 16, 16)] = jnp.right_shift(v, 16)


def _preload_packed(src_hbm, dst, s0):
    pltpu.sync_copy(src_hbm.at[pl.ds(s0 * PW, PSZ)], dst)


def _make_ea_kernel():
    scratch = [
        pltpu.VMEM((PSZ,), jnp.int32),
        pltpu.VMEM((PSZ,), jnp.int32),
        [pltpu.VMEM((CH,), jnp.int32) for _ in range(2)],
        [pltpu.VMEM((CH,), jnp.int32) for _ in range(2)],
        [pltpu.VMEM((CH, F), jnp.float32) for _ in range(2)],
        pltpu.VMEM_SHARED((NPAD, F), jnp.float32),
        [pltpu.SemaphoreType.DMA for _ in range(2)],
        [pltpu.SemaphoreType.DMA for _ in range(2)],
        [pltpu.SemaphoreType.DMA for _ in range(2)],
    ]

    def body(a_hbm, b_hbm, c_hbm, pc_hbm, pr_hbm, po,
             pcol, prow, idxc, idxr, bufab, accum, semb, sema, sems):
        cid = lax.axis_index("c")
        sid = lax.axis_index("s")
        w = sid * NC + cid
        row0 = sid * SROWS
        cnt, s0 = _worker_chunks(w)

        _init_accum(accum, bufab[0], row0, F)
        _preload_packed(pc_hbm, pcol, s0)
        _preload_packed(pr_hbm, prow, s0)
        plsc.subcore_barrier()

        def start(i, s):
            @pl.when(i >= s0 + 2)
            def _():
                pltpu.make_async_copy(bufab[s], accum.at[idxc[s]],
                                      sems[s]).wait()
            _unpack_idx(pcol, i - s0, idxc[s])
            _unpack_idx(prow, i - s0, idxr[s])
            pltpu.async_copy(c_hbm.at[pl.ds(i * CH, CH)], bufab[s], semb[s])

        def issue(i, s):
            pltpu.make_async_copy(c_hbm.at[pl.ds(i * CH, CH)], bufab[s],
                                  semb[s]).wait()
            pltpu.async_copy(a_hbm.at[idxc[s]], bufab[s], sema[s], add=True)
            pltpu.async_copy(b_hbm.at[idxr[s]], bufab[s], sema[s], add=True)

        def complete(i, s):
            pltpu.make_async_copy(a_hbm.at[idxc[s]], bufab[s],
                                  sema[s]).wait()
            pltpu.make_async_copy(b_hbm.at[idxr[s]], bufab[s],
                                  sema[s]).wait()

            @plsc.parallel_loop(0, CH, unroll=4)
            def _(r):
                for cc in range(F // 16):
                    sl = pl.ds(cc * 16, 16)
                    bufab[s][r, sl] = jnp.maximum(bufab[s][r, sl], 0.0)

            pltpu.async_copy(bufab[s], accum.at[idxc[s]], sems[s], add=True)

        _pipelined_chunks3(cnt, s0, start, issue, complete)
        for s in range(2):
            pltpu.make_async_copy(bufab[s], accum.at[idxc[s]],
                                  sems[s]).wait()

        plsc.subcore_barrier()
        _writeback(accum, bufab[0], po, row0, cid)

    return pl.kernel(
        body, out_type=jax.ShapeDtypeStruct((NC * NPAD, F), jnp.float32),
        mesh=_MESH, scratch_types=scratch)


def _make_hop_kernel():
    scratch = [
        pltpu.VMEM((PSZ,), jnp.int32),
        pltpu.VMEM((PSZ,), jnp.int32),
        [pltpu.VMEM((CH,), jnp.int32) for _ in range(2)],
        [pltpu.VMEM((CH,), jnp.int32) for _ in range(2)],
        [pltpu.VMEM((CH, F), jnp.float32) for _ in range(2)],
        pltpu.VMEM_SHARED((NPAD, F), jnp.float32),
        [pltpu.SemaphoreType.DMA for _ in range(2)],
        [pltpu.SemaphoreType.DMA for _ in range(2)],
    ]

    def body(t_hbm, pc_hbm, pr_hbm, po,
             pcol, prow, idxc, idxr, buf, accum, semb, sems):
        cid = lax.axis_index("c")
        sid = lax.axis_index("s")
        w = sid * NC + cid
        row0 = sid * SROWS
        cnt, s0 = _worker_chunks(w)

        _init_accum(accum, buf[0], row0, F)
        _preload_packed(pc_hbm, pcol, s0)
        _preload_packed(pr_hbm, prow, s0)
        plsc.subcore_barrier()

        def start(i, s):
            @pl.when(i >= s0 + 2)
            def _():
                pltpu.make_async_copy(buf[s], accum.at[idxc[s]],
                                      sems[s]).wait()
            _unpack_idx(pcol, i - s0, idxc[s])
            _unpack_idx(prow, i - s0, idxr[s])
            pltpu.async_copy(t_hbm.at[idxr[s]], buf[s], semb[s])

        def finish(i, s):
            pltpu.make_async_copy(t_hbm.at[idxr[s]], buf[s], semb[s]).wait()
            pltpu.async_copy(buf[s], accum.at[idxc[s]], sems[s], add=True)

        _pipelined_chunks(cnt, s0, start, finish)
        for s in range(2):
            pltpu.make_async_copy(buf[s], accum.at[idxc[s]], sems[s]).wait()

        plsc.subcore_barrier()
        _writeback(accum, buf[0], po, row0, cid)

    return pl.kernel(
        body, out_type=jax.ShapeDtypeStruct((NC * NPAD, F), jnp.float32),
        mesh=_MESH, scratch_types=scratch)


def _make_deg_kernel():
    DEGW = 128
    scratch = [
        pltpu.VMEM((PSZ,), jnp.int32),
        [pltpu.VMEM((CH,), jnp.int32) for _ in range(2)],
        pltpu.VMEM((CH, DEGW), jnp.float32),
        pltpu.VMEM_SHARED((NPAD, DEGW), jnp.float32),
        [pltpu.SemaphoreType.DMA for _ in range(2)],
    ]

    def body(pc_hbm, po, pcol, idxc, ones_v, dega, sems):
        cid = lax.axis_index("c")
        sid = lax.axis_index("s")
        w = sid * NC + cid
        row0 = sid * SROWS
        cnt, s0 = _worker_chunks(w)

        _init_accum(dega, ones_v, row0, DEGW)
        _preload_packed(pc_hbm, pcol, s0)

        def ob(r, _):
            for cc in range(DEGW // 16):
                ones_v[r, pl.ds(cc * 16, 16)] = jnp.ones((16,), jnp.float32)
            return 0
        lax.fori_loop(0, CH, ob, 0)

        plsc.subcore_barrier()

        def start(i, s):
            @pl.when(i >= s0 + 2)
            def _():
                pltpu.make_async_copy(ones_v, dega.at[idxc[s]],
                                      sems[s]).wait()
            _unpack_idx(pcol, i - s0, idxc[s])

        def finish(i, s):
            pltpu.async_copy(ones_v, dega.at[idxc[s]], sems[s], add=True)

        _pipelined_chunks(cnt, s0, start, finish)
        for s in range(2):
            pltpu.make_async_copy(ones_v, dega.at[idxc[s]], sems[s]).wait()

        plsc.subcore_barrier()
        _writeback(dega, ones_v, po, row0, cid)

    return pl.kernel(
        body, out_type=jax.ShapeDtypeStruct((NC * NPAD, 128), jnp.float32),
        mesh=_MESH, scratch_types=scratch)


_ea_call = _make_ea_kernel()
_hop_call = _make_hop_kernel()
_deg_call = _make_deg_kernel()


BLK = 1264
EBLK = 2000


def _mm(x, w):
    p = w.shape[1]

    def body(x_ref, w_ref, o_ref):
        o_ref[...] = jnp.dot(x_ref[...], w_ref[...],
                             preferred_element_type=jnp.float32)

    return pl.pallas_call(
        body, grid=(NPAD // BLK,),
        in_specs=[pl.BlockSpec((BLK, F), lambda i: (i, 0)),
                  pl.BlockSpec((F, p), lambda i: (0, 0))],
        out_specs=pl.BlockSpec((BLK, p), lambda i: (i, 0)),
        out_shape=jax.ShapeDtypeStruct((NPAD, p), jnp.float32))(x, w)


def _cmat(ea, w, b):
    def body(e_ref, w_ref, b_ref, o_ref):
        o_ref[...] = jnp.dot(e_ref[...], w_ref[...],
                             preferred_element_type=jnp.float32) + b_ref[...]

    return pl.pallas_call(
        body, grid=(E // EBLK,),
        in_specs=[pl.BlockSpec((EBLK, 16), lambda i: (i, 0)),
                  pl.BlockSpec((16, F), lambda i: (0, 0)),
                  pl.BlockSpec((1, F), lambda i: (0, 0))],
        out_specs=pl.BlockSpec((EBLK, F), lambda i: (i, 0)),
        out_shape=jax.ShapeDtypeStruct((E, F), jnp.float32))(ea, w, b)


def _ea_post_tag_first(p, degp, w2, b2, w0):
    def body(p_ref, dp_ref, w2_ref, b2_ref, w0_ref, o0_ref, t0_ref, dg_ref,
             ds_ref):
        deg = dp_ref[0] + dp_ref[1]
        dis = jnp.where(deg > 0, lax.rsqrt(jnp.maximum(deg, 1e-12)), 0.0)
        s = p_ref[0] + p_ref[1]
        h = jnp.dot(s, w2_ref[...], preferred_element_type=jnp.float32)
        h = h + deg * b2_ref[...]
        o0_ref[...] = jnp.dot(h, w0_ref[...],
                              preferred_element_type=jnp.float32)
        t0_ref[...] = dis * h
        dg_ref[...] = deg
        ds_ref[...] = dis

    return pl.pallas_call(
        body, grid=(NPAD // BLK,),
        in_specs=[pl.BlockSpec((2, BLK, F), lambda i: (0, i, 0)),
                  pl.BlockSpec((2, BLK, 1), lambda i: (0, i, 0)),
                  pl.BlockSpec((F, F), lambda i: (0, 0)),
                  pl.BlockSpec((1, F), lambda i: (0, 0)),
                  pl.BlockSpec((F, F), lambda i: (0, 0))],
        out_specs=[pl.BlockSpec((BLK, F), lambda i: (i, 0)),
                   pl.BlockSpec((BLK, F), lambda i: (i, 0)),
                   pl.BlockSpec((BLK, 1), lambda i: (i, 0)),
                   pl.BlockSpec((BLK, 1), lambda i: (i, 0))],
        out_shape=[jax.ShapeDtypeStruct((NPAD, F), jnp.float32),
                   jax.ShapeDtypeStruct((NPAD, F), jnp.float32),
                   jax.ShapeDtypeStruct((NPAD, 1), jnp.float32),
                   jax.ShapeDtypeStruct((NPAD, 1), jnp.float32)])(
                       p, degp, w2, b2, w0)


def _ea_post_tag_next(p, deg, dis, w2, b2, w0):
    def body(p_ref, dg_ref, ds_ref, w2_ref, b2_ref, w0_ref, o0_ref, t0_ref):
        s = p_ref[0] + p_ref[1]
        h = jnp.dot(s, w2_ref[...], preferred_element_type=jnp.float32)
        h = h + dg_ref[...] * b2_ref[...]
        o0_ref[...] = jnp.dot(h, w0_ref[...],
                              preferred_element_type=jnp.float32)
        t0_ref[...] = ds_ref[...] * h

    return pl.pallas_call(
        body, grid=(NPAD // BLK,),
        in_specs=[pl.BlockSpec((2, BLK, F), lambda i: (0, i, 0)),
                  pl.BlockSpec((BLK, 1), lambda i: (i, 0)),
                  pl.BlockSpec((BLK, 1), lambda i: (i, 0)),
                  pl.BlockSpec((F, F), lambda i: (0, 0)),
                  pl.BlockSpec((1, F), lambda i: (0, 0)),
                  pl.BlockSpec((F, F), lambda i: (0, 0))],
        out_specs=[pl.BlockSpec((BLK, F), lambda i: (i, 0)),
                   pl.BlockSpec((BLK, F), lambda i: (i, 0))],
        out_shape=[jax.ShapeDtypeStruct((NPAD, F), jnp.float32),
                   jax.ShapeDtypeStruct((NPAD, F), jnp.float32)])(
                       p, deg, dis, w2, b2, w0)


def _hopmix(p, dis, wk, outprev):
    def body(p_ref, ds_ref, wk_ref, op_ref, o_ref, t_ref):
        u = ds_ref[...] * (p_ref[0] + p_ref[1])
        o_ref[...] = op_ref[...] + jnp.dot(
            u, wk_ref[...], preferred_element_type=jnp.float32)
        t_ref[...] = ds_ref[...] * u

    return pl.pallas_call(
        body, grid=(NPAD // BLK,),
        in_specs=[pl.BlockSpec((2, BLK, F), lambda i: (0, i, 0)),
                  pl.BlockSpec((BLK, 1), lambda i: (i, 0)),
                  pl.BlockSpec((F, F), lambda i: (0, 0)),
                  pl.BlockSpec((BLK, F), lambda i: (i, 0))],
        out_specs=[pl.BlockSpec((BLK, F), lambda i: (i, 0)),
                   pl.BlockSpec((BLK, F), lambda i: (i, 0))],
        out_shape=[jax.ShapeDtypeStruct((NPAD, F), jnp.float32),
                   jax.ShapeDtypeStruct((NPAD, F), jnp.float32)])(
                       p, dis, wk, outprev)


def _hopmix_last_relu_mm(p, dis, wk, outprev, bc, wab):
    def body(p_ref, ds_ref, wk_ref, op_ref, bc_ref, wab_ref, ab_ref):
        u = ds_ref[...] * (p_ref[0] + p_ref[1])
        o = op_ref[...] + jnp.dot(u, wk_ref[...],
                                  preferred_element_type=jnp.float32)
        r = jnp.maximum(o + bc_ref[...], 0.0)
        ab_ref[...] = jnp.dot(r, wab_ref[...],
                              preferred_element_type=jnp.float32)

    return pl.pallas_call(
        body, grid=(NPAD // BLK,),
        in_specs=[pl.BlockSpec((2, BLK, F), lambda i: (0, i, 0)),
                  pl.BlockSpec((BLK, 1), lambda i: (i, 0)),
                  pl.BlockSpec((F, F), lambda i: (0, 0)),
                  pl.BlockSpec((BLK, F), lambda i: (i, 0)),
                  pl.BlockSpec((1, F), lambda i: (0, 0)),
                  pl.BlockSpec((F, 2 * F), lambda i: (0, 0))],
        out_specs=pl.BlockSpec((BLK, 2 * F), lambda i: (i, 0)),
        out_shape=jax.ShapeDtypeStruct((NPAD, 2 * F), jnp.float32))(
            p, dis, wk, outprev, bc, wab)


def _hopmix_bias(p, dis, wk, outprev, bc):
    def body(p_ref, ds_ref, wk_ref, op_ref, bc_ref, o_ref):
        u = ds_ref[...] * (p_ref[0] + p_ref[1])
        o_ref[...] = op_ref[...] + jnp.dot(
            u, wk_ref[...], preferred_element_type=jnp.float32) + bc_ref[...]

    return pl.pallas_call(
        body, grid=(NPAD // BLK,),
        in_specs=[pl.BlockSpec((2, BLK, F), lambda i: (0, i, 0)),
                  pl.BlockSpec((BLK, 1), lambda i: (i, 0)),
                  pl.BlockSpec((F, F), lambda i: (0, 0)),
                  pl.BlockSpec((BLK, F), lambda i: (i, 0)),
                  pl.BlockSpec((1, F), lambda i: (0, 0))],
        out_specs=pl.BlockSpec((BLK, F), lambda i: (i, 0)),
        out_shape=jax.ShapeDtypeStruct((NPAD, F), jnp.float32))(
            p, dis, wk, outprev, bc)


def kernel(x, edge_index, edge_attr, ea1_W1, ea1_b1, ea1_W2, ea1_b2,
           ea2_W1, ea2_b1, ea2_W2, ea2_b2, conv0_W, conv0_b,
           conv1_W, conv1_b, conv2_W, conv2_b):
    pr = _pack_idx(edge_index[0])
    pc = _pack_idx(edge_index[1])
    xf = jnp.pad(x[:, 4:4 + F], ((0, NPAD - N), (0, 0)))

    w1ab_1 = jnp.concatenate([ea1_W1[:F], ea1_W1[F:2 * F]], axis=1)
    w1ab_2 = jnp.concatenate([ea2_W1[:F], ea2_W1[F:2 * F]], axis=1)
    c1 = _cmat(edge_attr, ea1_W1[2 * F:], ea1_b1.reshape(1, F))
    c2 = _cmat(edge_attr, ea2_W1[2 * F:], ea2_b1.reshape(1, F))
    degp = _deg_call(pc)

    ab = _mm(xf, w1ab_1)
    po = _ea_call(ab[:, :F], ab[:, F:], c1, pc, pr)
    outp, t, deg, dis = _ea_post_tag_first(
        po.reshape(NC, NPAD, F), degp.reshape(NC, NPAD, F)[:, :, 0:1],
        ea1_W2, ea1_b2.reshape(1, F), conv0_W[0])
    for k in (1, 2):
        pk = _hop_call(t, pc, pr).reshape(NC, NPAD, F)
        outp, t = _hopmix(pk, dis, conv0_W[k], outp)
    pk = _hop_call(t, pc, pr).reshape(NC, NPAD, F)
    ab = _hopmix_last_relu_mm(pk, dis, conv0_W[3], outp,
                              conv0_b.reshape(1, F), w1ab_2)

    po = _ea_call(ab[:, :F], ab[:, F:], c2, pc, pr)
    outp, t = _ea_post_tag_next(po.reshape(NC, NPAD, F), deg, dis,
                                ea2_W2, ea2_b2.reshape(1, F), conv1_W[0])
    for k in (1, 2):
        pk = _hop_call(t, pc, pr).reshape(NC, NPAD, F)
        outp, t = _hopmix(pk, dis, conv1_W[k], outp)
    pk = _hop_call(t, pc, pr).reshape(NC, NPAD, F)
    ab = _hopmix_last_relu_mm(pk, dis, conv1_W[3], outp,
                              conv1_b.reshape(1, F), w1ab_2)

    po = _ea_call(ab[:, :F], ab[:, F:], c2, pc, pr)
    outp, t = _ea_post_tag_next(po.reshape(NC, NPAD, F), deg, dis,
                                ea2_W2, ea2_b2.reshape(1, F), conv2_W[0])
    for k in (1, 2):
        pk = _hop_call(t, pc, pr).reshape(NC, NPAD, F)
        outp, t = _hopmix(pk, dis, conv2_W[k], outp)
    pk = _hop_call(t, pc, pr).reshape(NC, NPAD, F)
    out = _hopmix_bias(pk, dis, conv2_W[3], outp, conv2_b.reshape(1, F))
    return out[:N]
